# baseline (speedup 1.0000x reference)
import functools

import jax
import jax.numpy as jnp
from jax.experimental import pallas as pl
from jax.experimental.pallas import tpu as pltpu

_BATCH_BLOCK = 2048


def _mse_kernel(batch, scale, lab_ref, h_ref, c_ref, out_ref, acc_ref):
    i = pl.program_id(0)
    n = pl.num_programs(0)

    @pl.when(i == 0)
    def _init():
        acc_ref[0] = 0.0

    row = lab_ref[0] % 8
    c = c_ref[pl.ds(row, 1), :]
    x = h_ref[...]
    sumsq = jnp.sum(x * x)
    colsum = jnp.sum(x, axis=0, keepdims=True)
    cross = jnp.sum(colsum * c)
    acc_ref[0] += sumsq - 2.0 * cross

    @pl.when(i == n - 1)
    def _fin():
        csq = jnp.sum(c * c)
        out_ref[0, 0] = (acc_ref[0] + batch * csq) * scale


def kernel(h, labels, centers):
    B, D = h.shape
    lab = jnp.asarray(labels, dtype=jnp.int32).reshape((1,))
    grid = (B // _BATCH_BLOCK,)
    total = pl.pallas_call(
        functools.partial(_mse_kernel, float(B), 1.0 / (B * D)),
        grid_spec=pltpu.PrefetchScalarGridSpec(
            num_scalar_prefetch=1,
            grid=grid,
            in_specs=[
                pl.BlockSpec((_BATCH_BLOCK, D), lambda i, lab_ref: (i, 0)),
                pl.BlockSpec((8, D), lambda i, lab_ref: (lab_ref[0] // 8, 0)),
            ],
            out_specs=pl.BlockSpec(
                (1, 1), lambda i, lab_ref: (0, 0), memory_space=pltpu.SMEM
            ),
            scratch_shapes=[pltpu.SMEM((1,), jnp.float32)],
        ),
        out_shape=jax.ShapeDtypeStruct((1, 1), jnp.float32),
    )(lab, h, centers)
    return jnp.reshape(total, ())


# speedup vs baseline: 1.0084x; 1.0084x over previous
import functools

import jax
import jax.numpy as jnp
from jax.experimental import pallas as pl
from jax.experimental.pallas import tpu as pltpu

_BATCH_BLOCK = 2048


def _mse_kernel(batch, scale, lab_ref, h_ref, c_hbm, out_ref,
                cvec_ref, colacc_ref, acc_ref, sem):
    i = pl.program_id(0)
    n = pl.num_programs(0)
    row_copy = pltpu.make_async_copy(
        c_hbm.at[pl.ds(lab_ref[0], 1), :], cvec_ref, sem
    )

    @pl.when(i == 0)
    def _init():
        acc_ref[0] = 0.0
        row_copy.start()

    x = h_ref[...]
    acc_ref[0] += jnp.sum(x * x)
    cs = jnp.sum(x, axis=0, keepdims=True)

    @pl.when(i == 0)
    def _first():
        colacc_ref[...] = cs

    @pl.when(i > 0)
    def _rest():
        colacc_ref[...] += cs

    @pl.when(i == n - 1)
    def _fin():
        row_copy.wait()
        c = cvec_ref[...]
        cross = jnp.sum(colacc_ref[...] * c)
        csq = jnp.sum(c * c)
        out_ref[0, 0] = (acc_ref[0] - 2.0 * cross + batch * csq) * scale


def kernel(h, labels, centers):
    B, D = h.shape
    lab = jnp.asarray(labels, dtype=jnp.int32).reshape((1,))
    grid = (B // _BATCH_BLOCK,)
    total = pl.pallas_call(
        functools.partial(_mse_kernel, float(B), 1.0 / (B * D)),
        grid_spec=pltpu.PrefetchScalarGridSpec(
            num_scalar_prefetch=1,
            grid=grid,
            in_specs=[
                pl.BlockSpec((_BATCH_BLOCK, D), lambda i, lab_ref: (i, 0)),
                pl.BlockSpec(memory_space=pl.ANY),
            ],
            out_specs=pl.BlockSpec(
                (1, 1), lambda i, lab_ref: (0, 0), memory_space=pltpu.SMEM
            ),
            scratch_shapes=[
                pltpu.VMEM((1, 64), jnp.float32),
                pltpu.VMEM((1, 64), jnp.float32),
                pltpu.SMEM((1,), jnp.float32),
                pltpu.SemaphoreType.DMA,
            ],
        ),
        out_shape=jax.ShapeDtypeStruct((1, 1), jnp.float32),
    )(lab, h, centers)
    return jnp.reshape(total, ())
